# baseline (speedup 1.0000x reference)
import jax
import jax.numpy as jnp
from jax.experimental import pallas as pl
from jax.experimental.pallas import tpu as pltpu

_BM = 200


def _fused_body(eye_ref, x_ref, w_ref, adj_ref, b_ref, o_ref, tmp_ref):
    i = pl.program_id(0)

    @pl.when(i == 0)
    def _():
        tmp_ref[...] = jnp.dot(x_ref[...], w_ref[...],
                               preferred_element_type=jnp.float32)

    acc = jnp.dot(adj_ref[...], tmp_ref[...],
                  preferred_element_type=jnp.float32)
    bm = o_ref.shape[0]
    bias = b_ref[...]
    eye = eye_ref[0]

    @pl.when(eye == 0)
    def _():
        o_ref[...] = acc + bias

    @pl.when(eye != 0)
    def _():
        o_ref[...] = tmp_ref[pl.ds(i * bm, bm), :] + bias


def kernel(input, adj, shared_weight, output_bias, eye):
    n, f_in = input.shape
    f_out = shared_weight.shape[-1]
    bm = _BM
    eye_arr = jnp.asarray(eye, jnp.int32).reshape(1)
    bias2d = output_bias.reshape(1, f_out)

    return pl.pallas_call(
        _fused_body,
        grid=(n // bm,),
        in_specs=[
            pl.BlockSpec(memory_space=pltpu.SMEM),
            pl.BlockSpec((n, f_in), lambda i: (0, 0)),
            pl.BlockSpec((f_in, f_out), lambda i: (0, 0)),
            pl.BlockSpec((bm, n), lambda i: (i, 0)),
            pl.BlockSpec((1, f_out), lambda i: (0, 0)),
        ],
        out_specs=pl.BlockSpec((bm, f_out), lambda i: (i, 0)),
        out_shape=jax.ShapeDtypeStruct((n, f_out), jnp.float32),
        scratch_shapes=[pltpu.VMEM((n, f_out), jnp.float32)],
        compiler_params=pltpu.CompilerParams(
            dimension_semantics=("arbitrary",),
            vmem_limit_bytes=100 * 1024 * 1024),
    )(eye_arr, input, shared_weight, adj, bias2d)


# speedup vs baseline: 1.0795x; 1.0795x over previous
import jax
import jax.numpy as jnp
from jax.experimental import pallas as pl
from jax.experimental.pallas import tpu as pltpu

_BM = 400


def _fused_body(eye_ref, x_hbm, w_ref, adj_ref, b_ref, o_ref,
                tmp_ref, xv_ref, sem):
    i = pl.program_id(0)

    @pl.when(i == 0)
    def _():
        cp = pltpu.make_async_copy(x_hbm, xv_ref, sem)
        cp.start()
        cp.wait()
        tmp_ref[...] = jnp.dot(xv_ref[...], w_ref[...],
                               preferred_element_type=jnp.float32)

    acc = jnp.dot(adj_ref[...], tmp_ref[...],
                  preferred_element_type=jnp.float32)
    bm = o_ref.shape[0]
    bias = b_ref[...]
    eye = eye_ref[0]

    @pl.when(eye == 0)
    def _():
        o_ref[...] = acc + bias

    @pl.when(eye != 0)
    def _():
        o_ref[...] = tmp_ref[pl.ds(i * bm, bm), :] + bias


def kernel(input, adj, shared_weight, output_bias, eye):
    n, f_in = input.shape
    f_out = shared_weight.shape[-1]
    bm = _BM
    eye_arr = jnp.asarray(eye, jnp.int32).reshape(1)
    bias2d = output_bias.reshape(1, f_out)

    return pl.pallas_call(
        _fused_body,
        grid=(pl.cdiv(n, bm),),
        in_specs=[
            pl.BlockSpec(memory_space=pltpu.SMEM),
            pl.BlockSpec(memory_space=pl.ANY),
            pl.BlockSpec((f_in, f_out), lambda i: (0, 0)),
            pl.BlockSpec((bm, n), lambda i: (i, 0)),
            pl.BlockSpec((1, f_out), lambda i: (0, 0)),
        ],
        out_specs=pl.BlockSpec((bm, f_out), lambda i: (i, 0)),
        out_shape=jax.ShapeDtypeStruct((n, f_out), jnp.float32),
        scratch_shapes=[
            pltpu.VMEM((n, f_out), jnp.float32),
            pltpu.VMEM((n, f_in), jnp.float32),
            pltpu.SemaphoreType.DMA,
        ],
        compiler_params=pltpu.CompilerParams(
            dimension_semantics=("arbitrary",),
            vmem_limit_bytes=100 * 1024 * 1024),
    )(eye_arr, input, shared_weight, adj, bias2d)


# speedup vs baseline: 1.1102x; 1.0285x over previous
import jax
import jax.numpy as jnp
from jax.experimental import pallas as pl
from jax.experimental.pallas import tpu as pltpu

_BM = 400


def _fused_body(eye_ref, x_ref, w_ref, adjt_ref, adjb_ref, b_ref, o_ref,
                tmp_ref):
    i = pl.program_id(0)

    @pl.when(i == 0)
    def _():
        tmp_ref[...] = jnp.dot(x_ref[...], w_ref[...],
                               preferred_element_type=jnp.float32)

    hm = adjt_ref.shape[0]
    acc_t = jnp.dot(adjt_ref[...], tmp_ref[...],
                    preferred_element_type=jnp.float32)
    acc_b = jnp.dot(adjb_ref[...], tmp_ref[...],
                    preferred_element_type=jnp.float32)
    bm = o_ref.shape[0]
    bias = b_ref[...]
    eye = eye_ref[0]

    @pl.when(eye == 0)
    def _():
        o_ref[pl.ds(0, hm), :] = acc_t + bias
        o_ref[pl.ds(hm, hm), :] = acc_b + bias

    @pl.when(eye != 0)
    def _():
        o_ref[...] = tmp_ref[pl.ds(i * bm, bm), :] + bias


def kernel(input, adj, shared_weight, output_bias, eye):
    n, f_in = input.shape
    f_out = shared_weight.shape[-1]
    bm = _BM
    hm = bm // 2
    eye_arr = jnp.asarray(eye, jnp.int32).reshape(1)
    bias2d = output_bias.reshape(1, f_out)

    return pl.pallas_call(
        _fused_body,
        grid=(n // bm,),
        in_specs=[
            pl.BlockSpec(memory_space=pltpu.SMEM),
            pl.BlockSpec((n, f_in), lambda i: (0, 0)),
            pl.BlockSpec((f_in, f_out), lambda i: (0, 0)),
            pl.BlockSpec((hm, n), lambda i: (2 * i, 0)),
            pl.BlockSpec((hm, n), lambda i: (2 * i + 1, 0)),
            pl.BlockSpec((1, f_out), lambda i: (0, 0)),
        ],
        out_specs=pl.BlockSpec((bm, f_out), lambda i: (i, 0)),
        out_shape=jax.ShapeDtypeStruct((n, f_out), jnp.float32),
        scratch_shapes=[pltpu.VMEM((n, f_out), jnp.float32)],
        compiler_params=pltpu.CompilerParams(
            dimension_semantics=("arbitrary",),
            vmem_limit_bytes=100 * 1024 * 1024),
    )(eye_arr, input, shared_weight, adj, adj, bias2d)


# speedup vs baseline: 1.1212x; 1.0099x over previous
import jax
import jax.numpy as jnp
from jax.experimental import pallas as pl
from jax.experimental.pallas import tpu as pltpu

_BM = 400


def _fused_body(eye_ref, x_ref, w_ref, adj_ref, b_ref, o_ref, tmp_ref):
    i = pl.program_id(0)

    @pl.when(i == 0)
    def _():
        tmp_ref[...] = jnp.dot(x_ref[...], w_ref[...],
                               preferred_element_type=jnp.float32)

    acc = jnp.dot(adj_ref[...].astype(jnp.bfloat16),
                  tmp_ref[...].astype(jnp.bfloat16),
                  preferred_element_type=jnp.float32)
    bm = o_ref.shape[0]
    bias = b_ref[...]
    eye = eye_ref[0]

    @pl.when(eye == 0)
    def _():
        o_ref[...] = acc + bias

    @pl.when(eye != 0)
    def _():
        o_ref[...] = tmp_ref[pl.ds(i * bm, bm), :] + bias


def kernel(input, adj, shared_weight, output_bias, eye):
    n, f_in = input.shape
    f_out = shared_weight.shape[-1]
    bm = _BM
    eye_arr = jnp.asarray(eye, jnp.int32).reshape(1)
    bias2d = output_bias.reshape(1, f_out)

    return pl.pallas_call(
        _fused_body,
        grid=(n // bm,),
        in_specs=[
            pl.BlockSpec(memory_space=pltpu.SMEM),
            pl.BlockSpec((n, f_in), lambda i: (0, 0)),
            pl.BlockSpec((f_in, f_out), lambda i: (0, 0)),
            pl.BlockSpec((bm, n), lambda i: (i, 0)),
            pl.BlockSpec((1, f_out), lambda i: (0, 0)),
        ],
        out_specs=pl.BlockSpec((bm, f_out), lambda i: (i, 0)),
        out_shape=jax.ShapeDtypeStruct((n, f_out), jnp.float32),
        scratch_shapes=[pltpu.VMEM((n, f_out), jnp.float32)],
        compiler_params=pltpu.CompilerParams(
            dimension_semantics=("arbitrary",),
            vmem_limit_bytes=100 * 1024 * 1024),
    )(eye_arr, input, shared_weight, adj, bias2d)
